# baseline (device time: 370171 ns/iter reference)
import jax
import jax.numpy as jnp
from jax import lax
from jax.experimental import pallas as pl
from jax.experimental.pallas import tpu as pltpu

N_DEV = 16
M = 4096
N_OUT = 2048
CHUNK = M // N_DEV
HALF = N_OUT // 2
COMM_DTYPE = jnp.bfloat16

MESH = pl.DeviceIdType.MESH
N_STEPS = 2 * (N_DEV - 1)


def _body(x_ref, w_ref, sx_ref, sw_ref, out_ref,
          cw_ref, ccw_ref, cw_send_sems, cw_recv_sems,
          ccw_send_sems, ccw_recv_sems, cw_credit, ccw_credit):
    my = lax.axis_index("i")
    left = (my - 1) % N_DEV
    right = (my + 1) % N_DEV

    barrier = pltpu.get_barrier_semaphore()
    for nbr in (left, right):
        pl.semaphore_signal(barrier, inc=1, device_id=(nbr,),
                            device_id_type=MESH)
    pl.semaphore_wait(barrier, 2)

    scale = sx_ref[0] * sw_ref[0]
    partial = lax.dot_general(
        x_ref[:, :].astype(jnp.bfloat16),
        w_ref[:, :].astype(jnp.bfloat16),
        (((1,), (0,)), ((), ())),
        preferred_element_type=jnp.float32,
    )
    out_ref[:, :] = partial * scale

    def cw_rows(c):
        return (pl.ds(c * CHUNK, CHUNK), pl.ds(0, HALF))

    def ccw_rows(c):
        return (pl.ds(c * CHUNK, CHUNK), pl.ds(HALF, HALF))

    def mk_cw(g):
        return pltpu.make_async_remote_copy(
            src_ref=cw_ref.at[g % 2],
            dst_ref=cw_ref.at[(g + 1) % 2],
            send_sem=cw_send_sems.at[g % 2],
            recv_sem=cw_recv_sems.at[(g + 1) % 2],
            device_id=(right,),
            device_id_type=MESH,
        )

    def mk_ccw(g):
        return pltpu.make_async_remote_copy(
            src_ref=ccw_ref.at[g % 2],
            dst_ref=ccw_ref.at[(g + 1) % 2],
            send_sem=ccw_send_sems.at[g % 2],
            recv_sem=ccw_recv_sems.at[(g + 1) % 2],
            device_id=(left,),
            device_id_type=MESH,
        )

    cw_ref[0, :, :] = out_ref[cw_rows(my)].astype(COMM_DTYPE)
    ccw_ref[0, :, :] = out_ref[ccw_rows(my)].astype(COMM_DTYPE)
    mk_cw(0).start()
    mk_ccw(0).start()

    for g in range(N_STEPS):
        slot_r = (g + 1) % 2
        cw = mk_cw(g)
        ccw = mk_ccw(g)
        is_rs = g < N_DEV - 1
        finalize = g == N_DEV - 2
        if is_rs:
            c_cw = (my - g - 1) % N_DEV
            c_ccw = (my + g + 1) % N_DEV
            cw.wait_recv()
            acc_cw = (cw_ref[slot_r, :, :].astype(jnp.float32)
                      + out_ref[cw_rows(c_cw)])
            cw_ref[slot_r, :, :] = acc_cw.astype(COMM_DTYPE)
            ccw.wait_recv()
            acc_ccw = (ccw_ref[slot_r, :, :].astype(jnp.float32)
                       + out_ref[ccw_rows(c_ccw)])
            ccw_ref[slot_r, :, :] = acc_ccw.astype(COMM_DTYPE)
        else:
            t = g - (N_DEV - 1)
            c_cw = (my - t) % N_DEV
            c_ccw = (my + t) % N_DEV
            cw.wait_recv()
            ccw.wait_recv()
        cw.wait_send()
        ccw.wait_send()
        if g < N_STEPS - 1:
            pl.semaphore_signal(cw_credit, inc=1, device_id=(left,),
                                device_id_type=MESH)
            pl.semaphore_signal(ccw_credit, inc=1, device_id=(right,),
                                device_id_type=MESH)
            pl.semaphore_wait(cw_credit, 1)
            pl.semaphore_wait(ccw_credit, 1)
            mk_cw(g + 1).start()
            mk_ccw(g + 1).start()
        if finalize:
            out_ref[cw_rows(c_cw)] = acc_cw
            out_ref[ccw_rows(c_ccw)] = acc_ccw
        if not is_rs:
            out_ref[cw_rows(c_cw)] = cw_ref[slot_r, :, :].astype(jnp.float32)
            out_ref[ccw_rows(c_ccw)] = ccw_ref[slot_r, :, :].astype(jnp.float32)


def kernel(x, w_mat, scale_x, scale_w):
    return pl.pallas_call(
        _body,
        out_shape=jax.ShapeDtypeStruct((M, N_OUT), jnp.float32),
        in_specs=[
            pl.BlockSpec(memory_space=pltpu.VMEM),
            pl.BlockSpec(memory_space=pltpu.VMEM),
            pl.BlockSpec(memory_space=pltpu.SMEM),
            pl.BlockSpec(memory_space=pltpu.SMEM),
        ],
        out_specs=pl.BlockSpec(memory_space=pltpu.VMEM),
        scratch_shapes=[
            pltpu.VMEM((2, CHUNK, HALF), COMM_DTYPE),
            pltpu.VMEM((2, CHUNK, HALF), COMM_DTYPE),
            pltpu.SemaphoreType.DMA((2,)),
            pltpu.SemaphoreType.DMA((2,)),
            pltpu.SemaphoreType.DMA((2,)),
            pltpu.SemaphoreType.DMA((2,)),
            pltpu.SemaphoreType.REGULAR,
            pltpu.SemaphoreType.REGULAR,
        ],
        compiler_params=pltpu.CompilerParams(
            collective_id=0,
            vmem_limit_bytes=100 * 1024 * 1024,
        ),
    )(x, w_mat, scale_x, scale_w)
